# baseline (speedup 1.0000x reference)
import functools

import jax
import jax.numpy as jnp
from jax import lax
from jax.experimental import pallas as pl
from jax.experimental.pallas import tpu as pltpu
from jax.experimental.pallas import tpu_sc as plsc

NUM_CORES = 2
NUM_SUBCORES = 16
NUM_WORKERS = NUM_CORES * NUM_SUBCORES

CH = 640
NBUF = 4
BK = 8192


def _tc_repack(V, D):
    grid = pl.cdiv(V, BK)

    def body(tT_ref, out_ref):
        x = tT_ref[...]
        y = jnp.swapaxes(x, 0, 1)
        y32 = y.reshape(BK // 4, 4, D)
        out_ref[...] = jnp.concatenate(
            [y32[:, jm, :] for jm in range(4)], axis=1
        )

    return pl.pallas_call(
        body,
        grid=(grid,),
        in_specs=[pl.BlockSpec((D, BK), lambda g: (0, g))],
        out_specs=pl.BlockSpec((BK * D // 128, 128), lambda g: (g, 0)),
        out_shape=jax.ShapeDtypeStruct((V * D // 128, 128), jnp.float32),
    )


def _make_gather(S, T, V, D):
    B = S * T
    k_per_w = (B // CH) // NUM_WORKERS
    b_per_w = k_per_w * CH
    assert b_per_w * NUM_WORKERS == B
    assert k_per_w % NBUF == 0 and k_per_w >= 3 * NBUF
    n_steps = k_per_w // NBUF

    mesh = plsc.VectorSubcoreMesh(core_axis_name="c", subcore_axis_name="s")

    scratch = (
        [pltpu.VMEM((b_per_w,), jnp.int32)]
        + [pltpu.VMEM((CH, D), jnp.float32) for _ in range(NBUF)]
        + [pltpu.SemaphoreType.DMA for _ in range(2 * NBUF)]
    )

    @functools.partial(
        pl.kernel,
        out_type=jax.ShapeDtypeStruct((B, D), jnp.float32),
        mesh=mesh,
        scratch_types=scratch,
        compiler_params=pltpu.CompilerParams(
            use_tc_tiling_on_sc=False, needs_layout_passes=False
        ),
    )
    def gather_kernel(idx_hbm, table_hbm, out_hbm, idx_v, *bufs):
        rows = bufs[:NBUF]
        gsem = bufs[NBUF : 2 * NBUF]
        ssem = bufs[2 * NBUF :]
        wid = lax.axis_index("s") * NUM_CORES + lax.axis_index("c")
        j0 = wid * b_per_w
        pltpu.sync_copy(idx_hbm.at[pl.ds(j0, b_per_w)], idx_v)

        def start_gather(c, b):
            pltpu.async_copy(
                table_hbm.at[idx_v.at[pl.ds(c * CH, CH)]], rows[b], gsem[b]
            )

        def wait_gather(b):
            pltpu.make_async_copy(
                table_hbm.at[idx_v.at[pl.ds(0, CH)]], rows[b], gsem[b]
            ).wait()

        def start_store(c, b):
            pltpu.async_copy(
                rows[b],
                out_hbm.at[pl.ds(j0 + c * CH, CH)],
                ssem[b],
            )

        def wait_store(b):
            pltpu.make_async_copy(
                rows[b],
                out_hbm.at[pl.ds(0, CH)],
                ssem[b],
            ).wait()

        for b in range(NBUF):
            start_gather(b, b)

        def step_body(step, carry):
            for b in range(NBUF):
                c = step * NBUF + b
                wait_gather(b)
                start_store(c, b)
                wait_store(b)
                start_gather(c + NBUF, b)
            return carry

        lax.fori_loop(0, n_steps - 1, step_body, 0)

        for b in range(NBUF):
            c = (n_steps - 1) * NBUF + b
            wait_gather(b)
            start_store(c, b)
        for b in range(NBUF):
            wait_store(b)

    return gather_kernel


def _tc_unpack(S, T, D):
    KP = S * D // 128

    def body(src_ref, out_ref):
        z = src_ref[...]
        z3 = z.reshape(KP, 4, D)
        o = jnp.concatenate(
            [jnp.swapaxes(z3[:, jm, :], 0, 1) for jm in range(4)], axis=1
        )
        out_ref[...] = o.reshape(1, D, S)

    return pl.pallas_call(
        body,
        grid=(T,),
        in_specs=[pl.BlockSpec((KP, 128), lambda g: (g, 0))],
        out_specs=pl.BlockSpec((1, D, S), lambda g: (g, 0, 0)),
        out_shape=jax.ShapeDtypeStruct((T, D, S), jnp.float32),
    )


def kernel(phonemes, table):
    S, T = phonemes.shape
    V, D = table.shape
    tableT = jnp.transpose(table)
    scr = _tc_repack(V, D)(tableT)
    table_lin = scr.reshape(-1).reshape(V, D)
    idx_flat = (
        phonemes.reshape(4, S // 4, T).transpose(2, 1, 0).reshape(-1)
    ).astype(jnp.int32)
    out2 = _make_gather(S, T, V, D)(idx_flat, table_lin)
    src = out2.reshape(-1).reshape(S * T * D // 128, 128)
    o3 = _tc_unpack(S, T, D)(src)
    return o3.transpose(2, 0, 1)


# speedup vs baseline: 1.0542x; 1.0542x over previous
import functools

import jax
import jax.numpy as jnp
from jax import lax
from jax.experimental import pallas as pl
from jax.experimental.pallas import tpu as pltpu
from jax.experimental.pallas import tpu_sc as plsc

NUM_CORES = 2
NUM_SUBCORES = 16
NUM_WORKERS = NUM_CORES * NUM_SUBCORES

CH = 640
NBUF = 4
BK = 8192


def _tc_repack(V, D):
    grid = pl.cdiv(V, BK)

    def body(tT_ref, out_ref):
        x = tT_ref[...]
        y = jnp.swapaxes(x, 0, 1)
        y32 = y.reshape(BK // 4, 4, D)
        out_ref[...] = jnp.concatenate(
            [y32[:, jm, :] for jm in range(4)], axis=1
        )

    return pl.pallas_call(
        body,
        grid=(grid,),
        in_specs=[pl.BlockSpec((D, BK), lambda g: (0, g))],
        out_specs=pl.BlockSpec((BK * D // 128, 128), lambda g: (g, 0)),
        out_shape=jax.ShapeDtypeStruct((V * D // 128, 128), jnp.float32),
    )


def _make_gather(S, T, V, D):
    B = S * T
    k_per_w = (B // CH) // NUM_WORKERS
    b_per_w = k_per_w * CH
    assert b_per_w * NUM_WORKERS == B
    assert k_per_w % NBUF == 0 and k_per_w >= 3 * NBUF
    n_steps = k_per_w // NBUF

    mesh = plsc.VectorSubcoreMesh(core_axis_name="c", subcore_axis_name="s")

    scratch = (
        [pltpu.VMEM((b_per_w,), jnp.int32)]
        + [pltpu.VMEM((CH, D), jnp.float32) for _ in range(NBUF)]
        + [pltpu.SemaphoreType.DMA for _ in range(2 * NBUF)]
    )

    @functools.partial(
        pl.kernel,
        out_type=jax.ShapeDtypeStruct((B, D), jnp.float32),
        mesh=mesh,
        scratch_types=scratch,
        compiler_params=pltpu.CompilerParams(
            use_tc_tiling_on_sc=False, needs_layout_passes=False
        ),
    )
    def gather_kernel(idx_hbm, table_hbm, out_hbm, idx_v, *bufs):
        rows = bufs[:NBUF]
        gsem = bufs[NBUF : 2 * NBUF]
        ssem = bufs[2 * NBUF :]
        wid = lax.axis_index("s") * NUM_CORES + lax.axis_index("c")
        j0 = wid * b_per_w
        pltpu.sync_copy(idx_hbm.at[pl.ds(j0, b_per_w)], idx_v)

        def start_gather(c, b):
            pltpu.async_copy(
                table_hbm.at[idx_v.at[pl.ds(c * CH, CH)]], rows[b], gsem[b]
            )

        def wait_gather(b):
            pltpu.make_async_copy(
                table_hbm.at[idx_v.at[pl.ds(0, CH)]], rows[b], gsem[b]
            ).wait()

        def start_store(c, b):
            pltpu.async_copy(
                rows[b],
                out_hbm.at[pl.ds(j0 + c * CH, CH)],
                ssem[b],
            )

        def wait_store(b):
            pltpu.make_async_copy(
                rows[b],
                out_hbm.at[pl.ds(0, CH)],
                ssem[b],
            ).wait()

        for b in range(NBUF):
            start_gather(b, b)

        def step_body(step, carry):
            for b in range(NBUF):
                c = step * NBUF + b
                wait_gather(b)
                start_store(c, b)
                wait_store(b)
                start_gather(c + NBUF, b)
            return carry

        lax.fori_loop(0, n_steps - 1, step_body, 0)

        for b in range(NBUF):
            c = (n_steps - 1) * NBUF + b
            wait_gather(b)
            start_store(c, b)
        for b in range(NBUF):
            wait_store(b)

    return gather_kernel


def kernel(phonemes, table):
    S, T = phonemes.shape
    V, D = table.shape
    tableT = jnp.transpose(table)
    scr = _tc_repack(V, D)(tableT)
    table_lin = scr.reshape(-1).reshape(V, D)
    idx_flat = phonemes.reshape(-1).astype(jnp.int32)
    out2 = _make_gather(S, T, V, D)(idx_flat, table_lin)
    return out2.reshape(S, T, D)
